# baseline (speedup 1.0000x reference)
import functools

import jax
import jax.numpy as jnp
from jax import lax
from jax.experimental import pallas as pl
from jax.experimental.pallas import tpu as pltpu
from jax.experimental.pallas import tpu_sc as plsc

_NC = 2
_NS = 16
_NW = _NC * _NS

_B, _L, _D = 4, 4096, 768
_N = _B * _L
_PER_W = _N // _NW
_CHUNK = 64
_NCHUNK = _PER_W // _CHUNK


@functools.partial(
    pl.kernel,
    mesh=plsc.VectorSubcoreMesh(core_axis_name="c", subcore_axis_name="s"),
    out_type=jax.ShapeDtypeStruct((_N, _D), jnp.float32),
    scratch_types=(
        [pltpu.VMEM((_CHUNK,), jnp.int32) for _ in range(_NCHUNK)]
        + [
            pltpu.VMEM((_CHUNK, _D), jnp.float32),
            pltpu.VMEM((_CHUNK, _D), jnp.float32),
            pltpu.SemaphoreType.DMA,
            pltpu.SemaphoreType.DMA,
            pltpu.SemaphoreType.DMA,
            pltpu.SemaphoreType.DMA,
            pltpu.SemaphoreType.DMA,
        ]
    ),
)
def _sc_gather(table_hbm, idx_hbm, out_hbm, *refs):
    idxs = refs[:_NCHUNK]
    rows0, rows1, isem, gsem0, gsem1, ssem0, ssem1 = refs[_NCHUNK:]
    wid = lax.axis_index("s") * _NC + lax.axis_index("c")
    base = wid * _PER_W
    icopies = [
        pltpu.async_copy(idx_hbm.at[wid, c], idxs[c], isem)
        for c in range(_NCHUNK)
    ]
    for cp in icopies:
        cp.wait()

    rows = (rows0, rows1)
    gsems = (gsem0, gsem1)
    ssems = (ssem0, ssem1)
    g = [None, None]
    s = [None, None]
    g[0] = pltpu.async_copy(table_hbm.at[idxs[0]], rows0, gsem0)
    for c in range(_NCHUNK):
        cur = c % 2
        g[cur].wait()
        s[cur] = pltpu.async_copy(
            rows[cur], out_hbm.at[pl.ds(base + c * _CHUNK, _CHUNK)], ssems[cur])
        if c + 1 < _NCHUNK:
            nxt = 1 - cur
            if s[nxt] is not None:
                s[nxt].wait()
            g[nxt] = pltpu.async_copy(
                table_hbm.at[idxs[c + 1]], rows[nxt], gsems[nxt])
    s[0].wait()
    s[1].wait()


def kernel(src, embedding_table):
    idx = src.reshape(_NW, _NCHUNK, _CHUNK).astype(jnp.int32)
    out = _sc_gather(embedding_table, idx)
    return out.reshape(_B, _L, _D)


# speedup vs baseline: 1.0530x; 1.0530x over previous
import functools

import jax
import jax.numpy as jnp
from jax import lax
from jax.experimental import pallas as pl
from jax.experimental.pallas import tpu as pltpu
from jax.experimental.pallas import tpu_sc as plsc

_NC = 2
_NS = 16
_NW = _NC * _NS

_B, _L, _D = 4, 4096, 768
_N = _B * _L
_PER_W = _N // _NW
_CHUNK = 32
_NCHUNK = _PER_W // _CHUNK
_NBUF = 4


@functools.partial(
    pl.kernel,
    mesh=plsc.VectorSubcoreMesh(core_axis_name="c", subcore_axis_name="s"),
    out_type=jax.ShapeDtypeStruct((_N, _D), jnp.float32),
    scratch_types=(
        [pltpu.VMEM((_NCHUNK, _CHUNK), jnp.int32)]
        + [pltpu.VMEM((_CHUNK, _D), jnp.float32) for _ in range(_NBUF)]
        + [pltpu.SemaphoreType.DMA for _ in range(2 * _NBUF)]
    ),
)
def _sc_gather(table_hbm, idx_hbm, out_hbm, idx_v, *refs):
    rows = refs[:_NBUF]
    gsems = refs[_NBUF:2 * _NBUF]
    ssems = refs[2 * _NBUF:]
    wid = lax.axis_index("s") * _NC + lax.axis_index("c")
    base = wid * _PER_W
    pltpu.sync_copy(idx_hbm.at[wid], idx_v)

    g = [None] * _NBUF
    s = [None] * _NBUF
    for p in range(_NBUF - 1):
        g[p] = pltpu.async_copy(table_hbm.at[idx_v.at[p]], rows[p], gsems[p])
    for c in range(_NCHUNK):
        b = c % _NBUF
        g[b].wait()
        s[b] = pltpu.async_copy(
            rows[b], out_hbm.at[pl.ds(base + c * _CHUNK, _CHUNK)], ssems[b])
        n = c + _NBUF - 1
        if n < _NCHUNK:
            nb = n % _NBUF
            if s[nb] is not None:
                s[nb].wait()
            g[nb] = pltpu.async_copy(
                table_hbm.at[idx_v.at[n]], rows[nb], gsems[nb])
    for b in range(_NBUF):
        s[b].wait()


def kernel(src, embedding_table):
    idx = src.reshape(_NW, _NCHUNK, _CHUNK).astype(jnp.int32)
    out = _sc_gather(embedding_table, idx)
    return out.reshape(_B, _L, _D)
